# baseline (speedup 1.0000x reference)
import functools

import jax
import jax.numpy as jnp
from jax import lax
from jax.experimental import pallas as pl
from jax.experimental.pallas import tpu as pltpu
from jax.experimental.pallas import tpu_sc as plsc

NC = 2
NS = 16
NW = NC * NS

NJ = 26
NI = 16384
NTI = NI // 128
NBLK = NJ * NTI
BPW = NBLK // NW
D = 32

mesh = plsc.VectorSubcoreMesh(core_axis_name="c", subcore_axis_name="s")


@functools.partial(
    pl.kernel,
    mesh=mesh,
    out_type=jax.ShapeDtypeStruct((NJ, 4, NTI, 8, 128), jnp.float32),
    scratch_types=[
        pltpu.VMEM((128,), jnp.int32),
        pltpu.VMEM((128,), jnp.int32),
        pltpu.VMEM((128, D), jnp.float32),
        pltpu.VMEM((128, D), jnp.float32),
        pltpu.VMEM((D, 128), jnp.float32),
        pltpu.VMEM((D, 128), jnp.float32),
        pltpu.SemaphoreType.DMA,
        pltpu.SemaphoreType.DMA,
        pltpu.SemaphoreType.DMA,
        pltpu.SemaphoreType.DMA,
    ],
    compiler_params=pltpu.CompilerParams(
        use_tc_tiling_on_sc=False, needs_layout_passes=False
    ),
)
def _emb(idxT, wt, out, ix0, ix1, g0, g1, t0, t1, sg0, sg1, sw0, sw1):
    wid = lax.axis_index("s") * NC + lax.axis_index("c")
    base = wid * BPW
    iota = lax.broadcasted_iota(jnp.int32, (16,), 0)

    def jti(k):
        bid = base + k
        j = bid // NTI
        return j, bid - j * NTI

    def fire_g(k, ixv, gv, sem):
        j, ti = jti(k)
        pltpu.sync_copy(idxT.at[j, pl.ds(ti * 128, 128)], ixv)
        pltpu.async_copy(wt.at[ixv], gv, sem)

    def wait_g(ixv, gv, sem):
        pltpu.make_async_copy(wt.at[ixv], gv, sem).wait()

    def transpose(gv, tv):
        def cbody(c, _):
            cvec = jnp.broadcast_to(c, (16,)).astype(jnp.int32)
            for l8 in range(8):
                v = plsc.load_gather(gv, [iota + l8 * 16, cvec])
                tv[c, pl.ds(l8 * 16, 16)] = v
            return ()

        lax.fori_loop(0, D, cbody, ())

    def w_copies(k, tv, sem):
        j, ti = jti(k)
        return [
            pltpu.make_async_copy(tv.at[pl.ds(g * 8, 8)], out.at[j, g, ti], sem)
            for g in range(4)
        ]

    def fire_w(k, tv, sem):
        for cpy in w_copies(k, tv, sem):
            cpy.start()

    def wait_w(k, tv, sem):
        for cpy in w_copies(k, tv, sem):
            cpy.wait()

    fire_g(0, ix0, g0, sg0)

    def body(k2, _):
        k = k2 * 2
        fire_g(k + 1, ix1, g1, sg1)
        wait_g(ix0, g0, sg0)

        @pl.when(k2 > 0)
        def _():
            wait_w(k - 2, t0, sw0)

        transpose(g0, t0)
        fire_w(k, t0, sw0)

        @pl.when(k2 < BPW // 2 - 1)
        def _():
            fire_g(k + 2, ix0, g0, sg0)

        wait_g(ix1, g1, sg1)

        @pl.when(k2 > 0)
        def _():
            wait_w(k - 1, t1, sw1)

        transpose(g1, t1)
        fire_w(k + 1, t1, sw1)
        return ()

    lax.fori_loop(0, BPW // 2, body, ())
    wait_w(BPW - 2, t0, sw0)
    wait_w(BPW - 1, t1, sw1)


def kernel(idx, weight):
    o = _emb(idx.T.astype(jnp.int32), weight)
    return o.transpose(2, 4, 0, 1, 3).reshape(NI, NJ, D)
